# baseline (speedup 1.0000x reference)
import functools

import jax
import jax.numpy as jnp
import numpy as np
from jax import lax
from jax.experimental import pallas as pl
from jax.experimental.pallas import tpu as pltpu
from jax.experimental.pallas import tpu_sc as plsc

N = 10000
E = 160000
NODE_DIM = 256
BOND_DIM = 16
HID = 1024
HEADS = 16
DH = HID // HEADS
MEMB = 64

PADN = 10240
NW = 32
EB = 40
NBLK = E // (EB * NW)
NCHUNK = 16
VW = 64

_PERM_T = np.array([(j % HEADS) * DH + j // HEADS for j in range(HID)],
                   dtype=np.int32)

_f32 = jnp.float32


def _node_proj(x, Wpx, Wpm_m, bp):
    Bn = 1000

    def body(x_ref, w_ref, mrow_ref, b_ref, o_ref):
        acc = jnp.dot(x_ref[...], w_ref[...], preferred_element_type=_f32)
        const = mrow_ref[...] + b_ref[...]
        o_ref[...] = acc + const

    return pl.pallas_call(
        body,
        grid=(N // Bn,),
        in_specs=[
            pl.BlockSpec((Bn, NODE_DIM), lambda i: (i, 0)),
            pl.BlockSpec((NODE_DIM, HID), lambda i: (0, 0)),
            pl.BlockSpec((1, HID), lambda i: (0, 0)),
            pl.BlockSpec((1, HID), lambda i: (0, 0)),
        ],
        out_specs=pl.BlockSpec((Bn, HID), lambda i: (i, 0)),
        out_shape=jax.ShapeDtypeStruct((N, HID), _f32),
    )(x, Wpx, Wpm_m, bp)


def _two_mm(h, Wa, ba, Wb, bb):
    Bn = 400

    def body(h_ref, wa_ref, ba_ref, wb_ref, bb_ref, oa_ref, ob_ref):
        hb = h_ref[...]
        oa_ref[...] = jnp.dot(hb, wa_ref[...], preferred_element_type=_f32) + ba_ref[...]
        ob_ref[...] = jnp.dot(hb, wb_ref[...], preferred_element_type=_f32) + bb_ref[...]

    return pl.pallas_call(
        body,
        grid=(N // Bn,),
        in_specs=[
            pl.BlockSpec((Bn, HID), lambda i: (i, 0)),
            pl.BlockSpec((HID, HID), lambda i: (0, 0)),
            pl.BlockSpec((1, HID), lambda i: (0, 0)),
            pl.BlockSpec((HID, HID), lambda i: (0, 0)),
            pl.BlockSpec((1, HID), lambda i: (0, 0)),
        ],
        out_specs=[
            pl.BlockSpec((Bn, HID), lambda i: (i, 0)),
            pl.BlockSpec((Bn, HID), lambda i: (i, 0)),
        ],
        out_shape=[
            jax.ShapeDtypeStruct((N, HID), _f32),
            jax.ShapeDtypeStruct((N, HID), _f32),
        ],
    )(h, Wa, ba, Wb, bb)


def _edge_proj(ea, We_r, be_r):
    Be = 2000

    def body(ea_ref, w_ref, b_ref, o_ref):
        o_ref[0] = (jnp.dot(ea_ref[...], w_ref[0],
                            preferred_element_type=_f32) + b_ref[0])

    return pl.pallas_call(
        body,
        grid=(E // Be, NCHUNK),
        in_specs=[
            pl.BlockSpec((Be, BOND_DIM), lambda i, f: (i, 0)),
            pl.BlockSpec((1, BOND_DIM, VW), lambda i, f: (f, 0, 0)),
            pl.BlockSpec((1, 1, VW), lambda i, f: (f, 0, 0)),
        ],
        out_specs=pl.BlockSpec((1, Be, VW), lambda i, f: (f, i, 0)),
        out_shape=jax.ShapeDtypeStruct((NCHUNK, E, VW), _f32),
    )(ea, We_r, be_r)


def _vchunks(v_t):
    return [v_t[:, f * VW:(f + 1) * VW] for f in range(NCHUNK)]


def _sc_pass_a(qt, kt, ec, dst3, src3):
    mesh = plsc.VectorSubcoreMesh(core_axis_name="c", subcore_axis_name="s")

    @functools.partial(
        pl.kernel,
        mesh=mesh,
        compiler_params=pltpu.CompilerParams(use_tc_tiling_on_sc=False),
        out_type=[
            jax.ShapeDtypeStruct((E, HEADS), _f32),
            jax.ShapeDtypeStruct((2, PADN, HEADS), _f32),
        ],
        scratch_types=[
            pltpu.VMEM((NBLK, EB), jnp.int32),
            pltpu.VMEM((NBLK, EB), jnp.int32),
            pltpu.VMEM((EB, HID), _f32),
            pltpu.VMEM((EB, HID), _f32),
            pltpu.VMEM((EB, VW), _f32),
            pltpu.VMEM((EB, VW), _f32),
            pltpu.VMEM((EB, HEADS), _f32),
            pltpu.VMEM((EB, HEADS), _f32),
            pltpu.VMEM_SHARED((PADN, HEADS), _f32),
            pltpu.SemaphoreType.DMA,
            pltpu.SemaphoreType.DMA,
            pltpu.SemaphoreType.DMA,
            pltpu.SemaphoreType.DMA,
        ],
    )
    def kern(qt_h, kt_h, ec_h, dst3_h, src3_h, ex_h, den_h,
             idx_d, idx_s, qr, kr, er0, er1, aacc, exb, den_acc,
             sem1, sem2, sem3, sem4):
        cid = lax.axis_index("c")
        sid = lax.axis_index("s")
        wid = sid * 2 + cid

        def zb(j, _):
            exb[j, :] = jnp.zeros((HEADS,), _f32)
            return 0
        lax.fori_loop(0, EB, zb, 0)
        for r in range(PADN // 16 // EB):
            pltpu.sync_copy(exb, den_acc.at[pl.ds(sid * (PADN // 16) + r * EB, EB)])
        pltpu.sync_copy(dst3_h.at[:, wid], idx_d)
        pltpu.sync_copy(src3_h.at[:, wid], idx_s)
        plsc.subcore_barrier()

        def blk(g, _):
            e0 = (g * NW + wid) * EB
            cq = pltpu.async_copy(qt_h.at[idx_d.at[g]], qr, sem1)
            ck = pltpu.async_copy(kt_h.at[idx_s.at[g]], kr, sem2)
            ers = (er0, er1)
            sems = (sem3, sem4)
            ce = pltpu.async_copy(ec_h.at[0, pl.ds(e0, EB)], er0, sem3)

            def za(j, _):
                aacc[j, :] = jnp.zeros((HEADS,), _f32)
                return 0
            lax.fori_loop(0, EB, za, 0)
            cq.wait()
            ck.wait()

            for f in range(NCHUNK):
                ce.wait()
                if f < NCHUNK - 1:
                    ce = pltpu.async_copy(ec_h.at[f + 1, pl.ds(e0, EB)],
                                          ers[(f + 1) % 2], sems[(f + 1) % 2])
                erf = ers[f % 2]

                def edge(j, _):
                    acc = aacc[j, :]
                    for c4 in range(DH // NCHUNK):
                        c = f * (DH // NCHUNK) + c4
                        qv = qr[j, pl.ds(c * 16, 16)]
                        kv = kr[j, pl.ds(c * 16, 16)]
                        ev = erf[j, pl.ds(c4 * 16, 16)]
                        acc = acc + qv * (kv + ev)
                    aacc[j, :] = acc
                    return 0
                lax.fori_loop(0, EB, edge, 0, unroll=2)

            def fin(j, _):
                exb[j, :] = jnp.exp(aacc[j, :])
                return 0
            lax.fori_loop(0, EB, fin, 0)

            pltpu.sync_copy(exb, den_acc.at[idx_d.at[g]], add=True)
            pltpu.sync_copy(exb, ex_h.at[pl.ds(e0, EB)])
            return 0
        lax.fori_loop(0, NBLK, blk, 0)

        plsc.subcore_barrier()
        rows = PADN // 16
        pltpu.sync_copy(den_acc.at[pl.ds(sid * rows, rows)],
                        den_h.at[cid, pl.ds(sid * rows, rows)])

    return kern(qt, kt, ec, dst3, src3)


def _sc_pass_b(vcs, ec, ex, dst3, src3):
    mesh = plsc.VectorSubcoreMesh(core_axis_name="c", subcore_axis_name="s")

    @functools.partial(
        pl.kernel,
        mesh=mesh,
        compiler_params=pltpu.CompilerParams(use_tc_tiling_on_sc=False),
        out_type=jax.ShapeDtypeStruct((2, PADN, NCHUNK, VW), _f32),
        scratch_types=[
            pltpu.VMEM((NBLK, EB), jnp.int32),
            pltpu.VMEM((NBLK, EB), jnp.int32),
            pltpu.VMEM((2, EB, VW), _f32),
            pltpu.VMEM((2, EB, VW), _f32),
            pltpu.VMEM((EB, VW), _f32),
            pltpu.VMEM((2, EB, HEADS), _f32),
            pltpu.VMEM_SHARED((PADN, VW), _f32),
            pltpu.SemaphoreType.DMA,
            pltpu.SemaphoreType.DMA,
            pltpu.SemaphoreType.DMA,
            pltpu.SemaphoreType.DMA,
            pltpu.SemaphoreType.DMA,
            pltpu.SemaphoreType.DMA,
        ],
    )
    def kern(v0, v1, v2, v3, v4, v5, v6, v7, v8, v9, v10, v11, v12, v13, v14,
             v15, ec_h, ex_h, dst3_h, src3_h, num_h,
             idx_d, idx_s, vr, er, pr, exb, acc,
             semv0, semv1, seme0, seme1, semx0, semx1):
        cid = lax.axis_index("c")
        sid = lax.axis_index("s")
        wid = sid * 2 + cid
        vts = (v0, v1, v2, v3, v4, v5, v6, v7, v8, v9, v10, v11, v12, v13,
               v14, v15)
        pltpu.sync_copy(dst3_h.at[:, wid], idx_d)
        pltpu.sync_copy(src3_h.at[:, wid], idx_s)

        for f in range(NCHUNK):
            def zb(j, _):
                for cc in range(VW // 16):
                    pr[j, pl.ds(cc * 16, 16)] = jnp.zeros((16,), _f32)
                return 0
            lax.fori_loop(0, EB, zb, 0)
            for r in range(PADN // 16 // EB):
                pltpu.sync_copy(pr, acc.at[pl.ds(sid * (PADN // 16) + r * EB, EB)])
            plsc.subcore_barrier()

            semv = (semv0, semv1)
            seme = (seme0, seme1)
            semx = (semx0, semx1)

            def issue(g, b):
                e0 = (g * NW + wid) * EB
                pltpu.async_copy(vts[f].at[idx_s.at[g]], vr.at[b], semv[b])
                pltpu.async_copy(ec_h.at[f, pl.ds(e0, EB)], er.at[b], seme[b])
                pltpu.async_copy(ex_h.at[pl.ds(e0, EB)], exb.at[b], semx[b])

            def waitb(b):
                pltpu.make_async_copy(vts[f].at[idx_s.at[0]], vr.at[b],
                                      semv[b]).wait()
                pltpu.make_async_copy(ec_h.at[f, pl.ds(0, EB)], er.at[b],
                                      seme[b]).wait()
                pltpu.make_async_copy(ex_h.at[pl.ds(0, EB)], exb.at[b],
                                      semx[b]).wait()

            def work(g, b):
                def edge(j, _):
                    exv = exb[b, j, :]
                    for cc in range(VW // 16):
                        sl = pl.ds(cc * 16, 16)
                        pr[j, sl] = (vr[b, j, sl] + er[b, j, sl]) * exv
                    return 0
                lax.fori_loop(0, EB, edge, 0, unroll=2)
                pltpu.sync_copy(pr, acc.at[idx_d.at[g]], add=True)

            issue(0, 0)

            def pair(p, _):
                g0 = 2 * p
                waitb(0)
                issue(g0 + 1, 1)
                work(g0, 0)
                waitb(1)
                issue(g0 + 2, 0)
                work(g0 + 1, 1)
                return 0
            lax.fori_loop(0, (NBLK - 1) // 2, pair, 0)
            waitb(0)
            work(NBLK - 1, 0)

            plsc.subcore_barrier()
            rows = PADN // 16
            pltpu.sync_copy(acc.at[pl.ds(sid * rows, rows)],
                            num_h.at[cid, pl.ds(sid * rows, rows), f])
            plsc.subcore_barrier()

    return kern(*vcs, ec, ex, dst3, src3)


def _finalize(num, den, skip_t, Wc1_p, bc1, Wc2_p, bc2_p, pka_r):
    Bn = 400

    def body(num_ref, den_ref, skip_ref, w1_ref, b1_ref, w2_ref, b2_ref,
             pka_ref, logit_ref, loss_ref):
        i = pl.program_id(0)
        nsum = num_ref[0] + num_ref[1]
        n_t = nsum.reshape(Bn, HID)
        d = den_ref[0] + den_ref[1] + 1e-16
        d_t = jnp.tile(d, (1, DH))
        h_out = n_t / d_t + skip_ref[...]
        hid1 = jnp.maximum(
            jnp.dot(h_out, w1_ref[...], preferred_element_type=_f32) + b1_ref[...],
            0.0)
        logits = jnp.dot(hid1, w2_ref[...], preferred_element_type=_f32) + b2_ref[...]
        logit_ref[...] = logits
        l0 = logits[:, 0:1]
        l1 = logits[:, 1:2]
        m = jnp.maximum(l0, l1)
        lz = m + jnp.log(jnp.exp(l0 - m) + jnp.exp(l1 - m))
        tgt = pka_ref[...] > 0.0
        picked = jnp.where(tgt, l1, l0)
        part = jnp.sum(lz - picked) * (1.0 / N)

        @pl.when(i == 0)
        def _():
            loss_ref[0, 0] = 0.0
        loss_ref[0, 0] += part

    return pl.pallas_call(
        body,
        grid=(N // Bn,),
        in_specs=[
            pl.BlockSpec((2, Bn, NCHUNK, VW), lambda i: (0, i, 0, 0)),
            pl.BlockSpec((2, Bn, HEADS), lambda i: (0, i, 0)),
            pl.BlockSpec((Bn, HID), lambda i: (i, 0)),
            pl.BlockSpec((HID, 256), lambda i: (0, 0)),
            pl.BlockSpec((1, 256), lambda i: (0, 0)),
            pl.BlockSpec((256, 128), lambda i: (0, 0)),
            pl.BlockSpec((1, 128), lambda i: (0, 0)),
            pl.BlockSpec((Bn, 1), lambda i: (i, 0)),
        ],
        out_specs=[
            pl.BlockSpec((Bn, 128), lambda i: (i, 0)),
            pl.BlockSpec(memory_space=pltpu.SMEM),
        ],
        out_shape=[
            jax.ShapeDtypeStruct((N, 128), _f32),
            jax.ShapeDtypeStruct((1, 1), _f32),
        ],
    )(num, den, skip_t, Wc1_p, bc1, Wc2_p, bc2_p, pka_r)


def kernel(x, edge_index, edge_attr, metal_id, batch_vec, pka_labels,
           metal_table, Wp, bp, Wq, bq, Wk, bk, Wv, bv, We, be,
           Wskip, bskip, Wc1, bc1, Wc2, bc2):
    perm = _PERM_T
    inv_scale = 1.0 / np.sqrt(DH)

    Wpx = Wp[:NODE_DIM]
    m_row = metal_table[metal_id]
    mrow_c = (jnp.zeros((8, MEMB), _f32).at[0:1].set(m_row))
    Wq_t = (Wq * inv_scale)[:, perm]
    bq_t = (bq * inv_scale)[perm][None, :]
    Wk_t = Wk[:, perm]
    bk_t = bk[perm][None, :]
    Wv_t = Wv[:, perm]
    bv_t = bv[perm][None, :]
    Wsk_t = Wskip[:, perm]
    bsk_t = bskip[perm][None, :]
    We_r = We[:, perm].reshape(BOND_DIM, NCHUNK, VW).transpose(1, 0, 2)
    be_r = be[perm].reshape(NCHUNK, 1, VW)
    Wc1_p = Wc1[perm, :]
    Wc2_p = jnp.zeros((256, 128), _f32).at[:, :2].set(Wc2)
    bc2_p = jnp.zeros((1, 128), _f32).at[0, :2].set(bc2)

    def mproj_body(m_ref, w_ref, o_ref):
        o_ref[...] = jnp.dot(m_ref[...], w_ref[...], preferred_element_type=_f32)

    mcontrib = pl.pallas_call(
        mproj_body,
        grid=(1,),
        in_specs=[pl.BlockSpec((8, MEMB), lambda i: (0, 0)),
                  pl.BlockSpec((MEMB, HID), lambda i: (0, 0))],
        out_specs=pl.BlockSpec((8, HID), lambda i: (0, 0)),
        out_shape=jax.ShapeDtypeStruct((8, HID), _f32),
    )(mrow_c, Wp[NODE_DIM:])[0:1]

    h = _node_proj(x, Wpx, mcontrib, bp[None, :])
    qt, kt = _two_mm(h, Wq_t, bq_t, Wk_t, bk_t)
    vt, skip_t = _two_mm(h, Wv_t, bv_t, Wsk_t, bsk_t)
    ec = _edge_proj(edge_attr, We_r, be_r)
    vcs = _vchunks(vt)

    src3 = edge_index[0].reshape(NBLK, NW, EB)
    dst3 = edge_index[1].reshape(NBLK, NW, EB)
    ex, den = _sc_pass_a(qt, kt, ec, dst3, src3)
    num = _sc_pass_b(vcs, ec, ex, dst3, src3)

    logits128, loss_acc = _finalize(
        num[:, :N], den[:, :N], skip_t, Wc1_p, bc1[None, :], Wc2_p, bc2_p,
        pka_labels[:, None])

    return (logits128[:, :2], loss_acc[0, 0])


# speedup vs baseline: 1.1229x; 1.1229x over previous
import functools

import jax
import jax.numpy as jnp
import numpy as np
from jax import lax
from jax.experimental import pallas as pl
from jax.experimental.pallas import tpu as pltpu
from jax.experimental.pallas import tpu_sc as plsc

N = 10000
E = 160000
NODE_DIM = 256
BOND_DIM = 16
HID = 1024
HEADS = 16
DH = HID // HEADS
MEMB = 64

PADN = 10240
NW = 32
EB = 40
NBLK = E // (EB * NW)
NCHUNK = 16
VW = 64

_PERM_T = np.array([(j % HEADS) * DH + j // HEADS for j in range(HID)],
                   dtype=np.int32)

_f32 = jnp.float32


def _node_proj(x, Wpx, Wpm_m, bp):
    Bn = 1000

    def body(x_ref, w_ref, mrow_ref, b_ref, o_ref):
        acc = jnp.dot(x_ref[...], w_ref[...], preferred_element_type=_f32)
        const = mrow_ref[...] + b_ref[...]
        o_ref[...] = acc + const

    return pl.pallas_call(
        body,
        grid=(N // Bn,),
        in_specs=[
            pl.BlockSpec((Bn, NODE_DIM), lambda i: (i, 0)),
            pl.BlockSpec((NODE_DIM, HID), lambda i: (0, 0)),
            pl.BlockSpec((1, HID), lambda i: (0, 0)),
            pl.BlockSpec((1, HID), lambda i: (0, 0)),
        ],
        out_specs=pl.BlockSpec((Bn, HID), lambda i: (i, 0)),
        out_shape=jax.ShapeDtypeStruct((N, HID), _f32),
    )(x, Wpx, Wpm_m, bp)


def _two_mm(h, Wa, ba, Wb, bb):
    Bn = 400

    def body(h_ref, wa_ref, ba_ref, wb_ref, bb_ref, oa_ref, ob_ref):
        hb = h_ref[...]
        oa_ref[...] = jnp.dot(hb, wa_ref[...], preferred_element_type=_f32) + ba_ref[...]
        ob_ref[...] = jnp.dot(hb, wb_ref[...], preferred_element_type=_f32) + bb_ref[...]

    return pl.pallas_call(
        body,
        grid=(N // Bn,),
        in_specs=[
            pl.BlockSpec((Bn, HID), lambda i: (i, 0)),
            pl.BlockSpec((HID, HID), lambda i: (0, 0)),
            pl.BlockSpec((1, HID), lambda i: (0, 0)),
            pl.BlockSpec((HID, HID), lambda i: (0, 0)),
            pl.BlockSpec((1, HID), lambda i: (0, 0)),
        ],
        out_specs=[
            pl.BlockSpec((Bn, HID), lambda i: (i, 0)),
            pl.BlockSpec((Bn, HID), lambda i: (i, 0)),
        ],
        out_shape=[
            jax.ShapeDtypeStruct((N, HID), _f32),
            jax.ShapeDtypeStruct((N, HID), _f32),
        ],
    )(h, Wa, ba, Wb, bb)


def _edge_proj(ea, We_r, be_r):
    Be = 2000

    def body(ea_ref, w_ref, b_ref, o_ref):
        o_ref[0] = (jnp.dot(ea_ref[...], w_ref[0],
                            preferred_element_type=_f32) + b_ref[0])

    return pl.pallas_call(
        body,
        grid=(E // Be, NCHUNK),
        in_specs=[
            pl.BlockSpec((Be, BOND_DIM), lambda i, f: (i, 0)),
            pl.BlockSpec((1, BOND_DIM, VW), lambda i, f: (f, 0, 0)),
            pl.BlockSpec((1, 1, VW), lambda i, f: (f, 0, 0)),
        ],
        out_specs=pl.BlockSpec((1, Be, VW), lambda i, f: (f, i, 0)),
        out_shape=jax.ShapeDtypeStruct((NCHUNK, E, VW), _f32),
    )(ea, We_r, be_r)


def _vchunks(v_t):
    return [v_t[:, f * VW:(f + 1) * VW] for f in range(NCHUNK)]


def _sc_pass_a(qt, kt, ec, dst3, src3):
    mesh = plsc.VectorSubcoreMesh(core_axis_name="c", subcore_axis_name="s")

    @functools.partial(
        pl.kernel,
        mesh=mesh,
        compiler_params=pltpu.CompilerParams(use_tc_tiling_on_sc=False),
        out_type=[
            jax.ShapeDtypeStruct((E, HEADS), _f32),
            jax.ShapeDtypeStruct((2, PADN, HEADS), _f32),
        ],
        scratch_types=[
            pltpu.VMEM((NBLK, EB), jnp.int32),
            pltpu.VMEM((NBLK, EB), jnp.int32),
            pltpu.VMEM((EB, HID), _f32),
            pltpu.VMEM((EB, HID), _f32),
            pltpu.VMEM((EB, VW), _f32),
            pltpu.VMEM((EB, VW), _f32),
            pltpu.VMEM((EB, HEADS), _f32),
            pltpu.VMEM((EB, HEADS), _f32),
            pltpu.VMEM_SHARED((PADN, HEADS), _f32),
            pltpu.SemaphoreType.DMA,
            pltpu.SemaphoreType.DMA,
            pltpu.SemaphoreType.DMA,
            pltpu.SemaphoreType.DMA,
        ],
    )
    def kern(qt_h, kt_h, ec_h, dst3_h, src3_h, ex_h, den_h,
             idx_d, idx_s, qr, kr, er0, er1, aacc, exb, den_acc,
             sem1, sem2, sem3, sem4):
        cid = lax.axis_index("c")
        sid = lax.axis_index("s")
        wid = sid * 2 + cid

        def zb(j, _):
            exb[j, :] = jnp.zeros((HEADS,), _f32)
            return 0
        lax.fori_loop(0, EB, zb, 0)
        for r in range(PADN // 16 // EB):
            pltpu.sync_copy(exb, den_acc.at[pl.ds(sid * (PADN // 16) + r * EB, EB)])
        pltpu.sync_copy(dst3_h.at[:, wid], idx_d)
        pltpu.sync_copy(src3_h.at[:, wid], idx_s)
        plsc.subcore_barrier()

        def blk(g, _):
            e0 = (g * NW + wid) * EB
            cq = pltpu.async_copy(qt_h.at[idx_d.at[g]], qr, sem1)
            ck = pltpu.async_copy(kt_h.at[idx_s.at[g]], kr, sem2)
            ers = (er0, er1)
            sems = (sem3, sem4)
            ce = pltpu.async_copy(ec_h.at[0, pl.ds(e0, EB)], er0, sem3)

            def za(j, _):
                aacc[j, :] = jnp.zeros((HEADS,), _f32)
                return 0
            lax.fori_loop(0, EB, za, 0)
            cq.wait()
            ck.wait()

            for f in range(NCHUNK):
                ce.wait()
                if f < NCHUNK - 1:
                    ce = pltpu.async_copy(ec_h.at[f + 1, pl.ds(e0, EB)],
                                          ers[(f + 1) % 2], sems[(f + 1) % 2])
                erf = ers[f % 2]

                def edge(j, _):
                    acc = aacc[j, :]
                    for c4 in range(DH // NCHUNK):
                        c = f * (DH // NCHUNK) + c4
                        qv = qr[j, pl.ds(c * 16, 16)]
                        kv = kr[j, pl.ds(c * 16, 16)]
                        ev = erf[j, pl.ds(c4 * 16, 16)]
                        acc = acc + qv * (kv + ev)
                    aacc[j, :] = acc
                    return 0
                lax.fori_loop(0, EB, edge, 0)

            def fin(j, _):
                exb[j, :] = jnp.exp(aacc[j, :])
                return 0
            lax.fori_loop(0, EB, fin, 0)

            pltpu.sync_copy(exb, den_acc.at[idx_d.at[g]], add=True)
            pltpu.sync_copy(exb, ex_h.at[pl.ds(e0, EB)])
            return 0
        lax.fori_loop(0, NBLK, blk, 0)

        plsc.subcore_barrier()
        rows = PADN // 16
        pltpu.sync_copy(den_acc.at[pl.ds(sid * rows, rows)],
                        den_h.at[cid, pl.ds(sid * rows, rows)])

    return kern(qt, kt, ec, dst3, src3)


def _sc_pass_b(vcs, ec, ex, dst3, src3):
    mesh = plsc.VectorSubcoreMesh(core_axis_name="c", subcore_axis_name="s")

    @functools.partial(
        pl.kernel,
        mesh=mesh,
        compiler_params=pltpu.CompilerParams(use_tc_tiling_on_sc=False),
        out_type=jax.ShapeDtypeStruct((2, PADN, NCHUNK, VW), _f32),
        scratch_types=[
            pltpu.VMEM((NBLK, EB), jnp.int32),
            pltpu.VMEM((NBLK, EB), jnp.int32),
            pltpu.VMEM((2, EB, VW), _f32),
            pltpu.VMEM((2, EB, VW), _f32),
            pltpu.VMEM((EB, VW), _f32),
            pltpu.VMEM((2, EB, HEADS), _f32),
            pltpu.VMEM_SHARED((PADN, VW), _f32),
            pltpu.SemaphoreType.DMA,
            pltpu.SemaphoreType.DMA,
            pltpu.SemaphoreType.DMA,
            pltpu.SemaphoreType.DMA,
            pltpu.SemaphoreType.DMA,
            pltpu.SemaphoreType.DMA,
        ],
    )
    def kern(v0, v1, v2, v3, v4, v5, v6, v7, v8, v9, v10, v11, v12, v13, v14,
             v15, ec_h, ex_h, dst3_h, src3_h, num_h,
             idx_d, idx_s, vr, er, pr, exb, acc,
             semv0, semv1, seme0, seme1, semx0, semx1):
        cid = lax.axis_index("c")
        sid = lax.axis_index("s")
        wid = sid * 2 + cid
        vts = (v0, v1, v2, v3, v4, v5, v6, v7, v8, v9, v10, v11, v12, v13,
               v14, v15)
        pltpu.sync_copy(dst3_h.at[:, wid], idx_d)
        pltpu.sync_copy(src3_h.at[:, wid], idx_s)

        for f in range(NCHUNK):
            def zb(j, _):
                for cc in range(VW // 16):
                    pr[j, pl.ds(cc * 16, 16)] = jnp.zeros((16,), _f32)
                return 0
            lax.fori_loop(0, EB, zb, 0)
            for r in range(PADN // 16 // EB):
                pltpu.sync_copy(pr, acc.at[pl.ds(sid * (PADN // 16) + r * EB, EB)])
            plsc.subcore_barrier()

            semv = (semv0, semv1)
            seme = (seme0, seme1)
            semx = (semx0, semx1)

            def issue(g, b):
                e0 = (g * NW + wid) * EB
                pltpu.async_copy(vts[f].at[idx_s.at[g]], vr.at[b], semv[b])
                pltpu.async_copy(ec_h.at[f, pl.ds(e0, EB)], er.at[b], seme[b])
                pltpu.async_copy(ex_h.at[pl.ds(e0, EB)], exb.at[b], semx[b])

            def waitb(b):
                pltpu.make_async_copy(vts[f].at[idx_s.at[0]], vr.at[b],
                                      semv[b]).wait()
                pltpu.make_async_copy(ec_h.at[f, pl.ds(0, EB)], er.at[b],
                                      seme[b]).wait()
                pltpu.make_async_copy(ex_h.at[pl.ds(0, EB)], exb.at[b],
                                      semx[b]).wait()

            def work(g, b):
                def edge(j, _):
                    exv = exb[b, j, :]
                    for cc in range(VW // 16):
                        sl = pl.ds(cc * 16, 16)
                        pr[j, sl] = (vr[b, j, sl] + er[b, j, sl]) * exv
                    return 0
                lax.fori_loop(0, EB, edge, 0)
                pltpu.sync_copy(pr, acc.at[idx_d.at[g]], add=True)

            issue(0, 0)

            def pair(p, _):
                g0 = 2 * p
                waitb(0)
                issue(g0 + 1, 1)
                work(g0, 0)
                waitb(1)
                issue(g0 + 2, 0)
                work(g0 + 1, 1)
                return 0
            lax.fori_loop(0, (NBLK - 1) // 2, pair, 0)
            waitb(0)
            work(NBLK - 1, 0)

            plsc.subcore_barrier()
            rows = PADN // 16
            pltpu.sync_copy(acc.at[pl.ds(sid * rows, rows)],
                            num_h.at[cid, pl.ds(sid * rows, rows), f])
            plsc.subcore_barrier()

    return kern(*vcs, ec, ex, dst3, src3)


def _finalize(num, den, skip_t, Wc1_p, bc1, Wc2_p, bc2_p, pka_r):
    Bn = 400

    def body(num_ref, den_ref, skip_ref, w1_ref, b1_ref, w2_ref, b2_ref,
             pka_ref, logit_ref, loss_ref):
        i = pl.program_id(0)
        nsum = num_ref[0] + num_ref[1]
        n_t = nsum.reshape(Bn, HID)
        d = den_ref[0] + den_ref[1] + 1e-16
        d_t = jnp.tile(d, (1, DH))
        h_out = n_t / d_t + skip_ref[...]
        hid1 = jnp.maximum(
            jnp.dot(h_out, w1_ref[...], preferred_element_type=_f32) + b1_ref[...],
            0.0)
        logits = jnp.dot(hid1, w2_ref[...], preferred_element_type=_f32) + b2_ref[...]
        logit_ref[...] = logits
        l0 = logits[:, 0:1]
        l1 = logits[:, 1:2]
        m = jnp.maximum(l0, l1)
        lz = m + jnp.log(jnp.exp(l0 - m) + jnp.exp(l1 - m))
        tgt = pka_ref[...] > 0.0
        picked = jnp.where(tgt, l1, l0)
        part = jnp.sum(lz - picked) * (1.0 / N)

        @pl.when(i == 0)
        def _():
            loss_ref[0, 0] = 0.0
        loss_ref[0, 0] += part

    return pl.pallas_call(
        body,
        grid=(N // Bn,),
        in_specs=[
            pl.BlockSpec((2, Bn, NCHUNK, VW), lambda i: (0, i, 0, 0)),
            pl.BlockSpec((2, Bn, HEADS), lambda i: (0, i, 0)),
            pl.BlockSpec((Bn, HID), lambda i: (i, 0)),
            pl.BlockSpec((HID, 256), lambda i: (0, 0)),
            pl.BlockSpec((1, 256), lambda i: (0, 0)),
            pl.BlockSpec((256, 128), lambda i: (0, 0)),
            pl.BlockSpec((1, 128), lambda i: (0, 0)),
            pl.BlockSpec((Bn, 1), lambda i: (i, 0)),
        ],
        out_specs=[
            pl.BlockSpec((Bn, 128), lambda i: (i, 0)),
            pl.BlockSpec(memory_space=pltpu.SMEM),
        ],
        out_shape=[
            jax.ShapeDtypeStruct((N, 128), _f32),
            jax.ShapeDtypeStruct((1, 1), _f32),
        ],
    )(num, den, skip_t, Wc1_p, bc1, Wc2_p, bc2_p, pka_r)


def kernel(x, edge_index, edge_attr, metal_id, batch_vec, pka_labels,
           metal_table, Wp, bp, Wq, bq, Wk, bk, Wv, bv, We, be,
           Wskip, bskip, Wc1, bc1, Wc2, bc2):
    perm = _PERM_T
    inv_scale = 1.0 / np.sqrt(DH)

    Wpx = Wp[:NODE_DIM]
    m_row = metal_table[metal_id]
    mrow_c = (jnp.zeros((8, MEMB), _f32).at[0:1].set(m_row))
    Wq_t = (Wq * inv_scale)[:, perm]
    bq_t = (bq * inv_scale)[perm][None, :]
    Wk_t = Wk[:, perm]
    bk_t = bk[perm][None, :]
    Wv_t = Wv[:, perm]
    bv_t = bv[perm][None, :]
    Wsk_t = Wskip[:, perm]
    bsk_t = bskip[perm][None, :]
    We_r = We[:, perm].reshape(BOND_DIM, NCHUNK, VW).transpose(1, 0, 2)
    be_r = be[perm].reshape(NCHUNK, 1, VW)
    Wc1_p = Wc1[perm, :]
    Wc2_p = jnp.zeros((256, 128), _f32).at[:, :2].set(Wc2)
    bc2_p = jnp.zeros((1, 128), _f32).at[0, :2].set(bc2)

    def mproj_body(m_ref, w_ref, o_ref):
        o_ref[...] = jnp.dot(m_ref[...], w_ref[...], preferred_element_type=_f32)

    mcontrib = pl.pallas_call(
        mproj_body,
        grid=(1,),
        in_specs=[pl.BlockSpec((8, MEMB), lambda i: (0, 0)),
                  pl.BlockSpec((MEMB, HID), lambda i: (0, 0))],
        out_specs=pl.BlockSpec((8, HID), lambda i: (0, 0)),
        out_shape=jax.ShapeDtypeStruct((8, HID), _f32),
    )(mrow_c, Wp[NODE_DIM:])[0:1]

    h = _node_proj(x, Wpx, mcontrib, bp[None, :])
    qt, kt = _two_mm(h, Wq_t, bq_t, Wk_t, bk_t)
    vt, skip_t = _two_mm(h, Wv_t, bv_t, Wsk_t, bsk_t)
    ec = _edge_proj(edge_attr, We_r, be_r)
    vcs = _vchunks(vt)

    src3 = edge_index[0].reshape(NBLK, NW, EB)
    dst3 = edge_index[1].reshape(NBLK, NW, EB)
    ex, den = _sc_pass_a(qt, kt, ec, dst3, src3)
    num = _sc_pass_b(vcs, ec, ex, dst3, src3)

    logits128, loss_acc = _finalize(
        num[:, :N], den[:, :N], skip_t, Wc1_p, bc1[None, :], Wc2_p, bc2_p,
        pka_labels[:, None])

    return (logits128[:, :2], loss_acc[0, 0])


# speedup vs baseline: 1.1253x; 1.0022x over previous
import functools

import jax
import jax.numpy as jnp
import numpy as np
from jax import lax
from jax.experimental import pallas as pl
from jax.experimental.pallas import tpu as pltpu
from jax.experimental.pallas import tpu_sc as plsc

N = 10000
E = 160000
NODE_DIM = 256
BOND_DIM = 16
HID = 1024
HEADS = 16
DH = HID // HEADS
MEMB = 64

PADN = 10240
NW = 32
EB = 40
NBLK = E // (EB * NW)
NCHUNK = 16
VW = 64

_PERM_T = np.array([(j % HEADS) * DH + j // HEADS for j in range(HID)],
                   dtype=np.int32)

_f32 = jnp.float32


def _node_proj(x, Wpx, Wpm_m, bp):
    Bn = 1000

    def body(x_ref, w_ref, mrow_ref, b_ref, o_ref):
        acc = jnp.dot(x_ref[...], w_ref[...], preferred_element_type=_f32)
        const = mrow_ref[...] + b_ref[...]
        o_ref[...] = acc + const

    return pl.pallas_call(
        body,
        grid=(N // Bn,),
        in_specs=[
            pl.BlockSpec((Bn, NODE_DIM), lambda i: (i, 0)),
            pl.BlockSpec((NODE_DIM, HID), lambda i: (0, 0)),
            pl.BlockSpec((1, HID), lambda i: (0, 0)),
            pl.BlockSpec((1, HID), lambda i: (0, 0)),
        ],
        out_specs=pl.BlockSpec((Bn, HID), lambda i: (i, 0)),
        out_shape=jax.ShapeDtypeStruct((N, HID), _f32),
    )(x, Wpx, Wpm_m, bp)


def _two_mm(h, Wa, ba, Wb, bb):
    Bn = 400

    def body(h_ref, wa_ref, ba_ref, wb_ref, bb_ref, oa_ref, ob_ref):
        hb = h_ref[...]
        oa_ref[...] = jnp.dot(hb, wa_ref[...], preferred_element_type=_f32) + ba_ref[...]
        ob_ref[...] = jnp.dot(hb, wb_ref[...], preferred_element_type=_f32) + bb_ref[...]

    return pl.pallas_call(
        body,
        grid=(N // Bn,),
        in_specs=[
            pl.BlockSpec((Bn, HID), lambda i: (i, 0)),
            pl.BlockSpec((HID, HID), lambda i: (0, 0)),
            pl.BlockSpec((1, HID), lambda i: (0, 0)),
            pl.BlockSpec((HID, HID), lambda i: (0, 0)),
            pl.BlockSpec((1, HID), lambda i: (0, 0)),
        ],
        out_specs=[
            pl.BlockSpec((Bn, HID), lambda i: (i, 0)),
            pl.BlockSpec((Bn, HID), lambda i: (i, 0)),
        ],
        out_shape=[
            jax.ShapeDtypeStruct((N, HID), _f32),
            jax.ShapeDtypeStruct((N, HID), _f32),
        ],
    )(h, Wa, ba, Wb, bb)


def _edge_proj(ea, We_r, be_r):
    Be = 2000

    def body(ea_ref, w_ref, b_ref, o_ref):
        o_ref[0] = (jnp.dot(ea_ref[...], w_ref[0],
                            preferred_element_type=_f32) + b_ref[0])

    return pl.pallas_call(
        body,
        grid=(E // Be, NCHUNK),
        in_specs=[
            pl.BlockSpec((Be, BOND_DIM), lambda i, f: (i, 0)),
            pl.BlockSpec((1, BOND_DIM, VW), lambda i, f: (f, 0, 0)),
            pl.BlockSpec((1, 1, VW), lambda i, f: (f, 0, 0)),
        ],
        out_specs=pl.BlockSpec((1, Be, VW), lambda i, f: (f, i, 0)),
        out_shape=jax.ShapeDtypeStruct((NCHUNK, E, VW), _f32),
    )(ea, We_r, be_r)


def _vchunks(v_t):
    return [v_t[:, f * VW:(f + 1) * VW] for f in range(NCHUNK)]


def _sc_pass_a(qt, kt, ec, dst3, src3):
    mesh = plsc.VectorSubcoreMesh(core_axis_name="c", subcore_axis_name="s")

    @functools.partial(
        pl.kernel,
        mesh=mesh,
        compiler_params=pltpu.CompilerParams(use_tc_tiling_on_sc=False),
        out_type=[
            jax.ShapeDtypeStruct((E, HEADS), _f32),
            jax.ShapeDtypeStruct((2, PADN, HEADS), _f32),
        ],
        scratch_types=[
            pltpu.VMEM((NBLK, EB), jnp.int32),
            pltpu.VMEM((NBLK, EB), jnp.int32),
            pltpu.VMEM((EB, HID), _f32),
            pltpu.VMEM((EB, HID), _f32),
            pltpu.VMEM((EB, VW), _f32),
            pltpu.VMEM((EB, VW), _f32),
            pltpu.VMEM((EB, HEADS), _f32),
            pltpu.VMEM((EB, HEADS), _f32),
            pltpu.VMEM_SHARED((PADN, HEADS), _f32),
            pltpu.SemaphoreType.DMA,
            pltpu.SemaphoreType.DMA,
            pltpu.SemaphoreType.DMA,
            pltpu.SemaphoreType.DMA,
        ],
    )
    def kern(qt_h, kt_h, ec_h, dst3_h, src3_h, ex_h, den_h,
             idx_d, idx_s, qr, kr, er0, er1, aacc, exb, den_acc,
             sem1, sem2, sem3, sem4):
        cid = lax.axis_index("c")
        sid = lax.axis_index("s")
        wid = sid * 2 + cid

        def zb(j, _):
            exb[j, :] = jnp.zeros((HEADS,), _f32)
            return 0
        lax.fori_loop(0, EB, zb, 0)
        for r in range(PADN // 16 // EB):
            pltpu.sync_copy(exb, den_acc.at[pl.ds(sid * (PADN // 16) + r * EB, EB)])
        pltpu.sync_copy(dst3_h.at[:, wid], idx_d)
        pltpu.sync_copy(src3_h.at[:, wid], idx_s)
        plsc.subcore_barrier()

        def blk(g, _):
            e0 = (g * NW + wid) * EB
            cq = pltpu.async_copy(qt_h.at[idx_d.at[g]], qr, sem1)
            ck = pltpu.async_copy(kt_h.at[idx_s.at[g]], kr, sem2)
            ers = (er0, er1)
            sems = (sem3, sem4)
            ce = pltpu.async_copy(ec_h.at[0, pl.ds(e0, EB)], er0, sem3)

            def za(j, _):
                aacc[j, :] = jnp.zeros((HEADS,), _f32)
                return 0
            lax.fori_loop(0, EB, za, 0)
            cq.wait()
            ck.wait()

            for f in range(NCHUNK):
                ce.wait()
                if f < NCHUNK - 1:
                    ce = pltpu.async_copy(ec_h.at[f + 1, pl.ds(e0, EB)],
                                          ers[(f + 1) % 2], sems[(f + 1) % 2])
                erf = ers[f % 2]

                def edge(j, _):
                    acc = aacc[j, :]
                    for c4 in range(DH // NCHUNK):
                        c = f * (DH // NCHUNK) + c4
                        qv = qr[j, pl.ds(c * 16, 16)]
                        kv = kr[j, pl.ds(c * 16, 16)]
                        ev = erf[j, pl.ds(c4 * 16, 16)]
                        acc = acc + qv * (kv + ev)
                    aacc[j, :] = acc
                    return 0
                lax.fori_loop(0, EB, edge, 0)

            def fin(j, _):
                exb[j, :] = jnp.exp(aacc[j, :])
                return 0
            lax.fori_loop(0, EB, fin, 0)

            pltpu.sync_copy(exb, den_acc.at[idx_d.at[g]], add=True)
            pltpu.sync_copy(exb, ex_h.at[pl.ds(e0, EB)])
            return 0
        lax.fori_loop(0, NBLK, blk, 0)

        plsc.subcore_barrier()
        rows = PADN // 16
        pltpu.sync_copy(den_acc.at[pl.ds(sid * rows, rows)],
                        den_h.at[cid, pl.ds(sid * rows, rows)])

    return kern(qt, kt, ec, dst3, src3)


def _sc_pass_b(vcs, ec, ex, dst3, src3):
    mesh = plsc.VectorSubcoreMesh(core_axis_name="c", subcore_axis_name="s")

    @functools.partial(
        pl.kernel,
        mesh=mesh,
        compiler_params=pltpu.CompilerParams(use_tc_tiling_on_sc=False),
        out_type=jax.ShapeDtypeStruct((2, PADN, NCHUNK, VW), _f32),
        scratch_types=[
            pltpu.VMEM((NBLK, EB), jnp.int32),
            pltpu.VMEM((NBLK, EB), jnp.int32),
            pltpu.VMEM((2, EB, VW), _f32),
            pltpu.VMEM((2, EB, VW), _f32),
            pltpu.VMEM((2, EB, VW), _f32),
            pltpu.VMEM((2, EB, HEADS), _f32),
            pltpu.VMEM_SHARED((PADN, VW), _f32),
            pltpu.SemaphoreType.DMA,
            pltpu.SemaphoreType.DMA,
            pltpu.SemaphoreType.DMA,
            pltpu.SemaphoreType.DMA,
            pltpu.SemaphoreType.DMA,
            pltpu.SemaphoreType.DMA,
            pltpu.SemaphoreType.DMA,
            pltpu.SemaphoreType.DMA,
        ],
    )
    def kern(v0, v1, v2, v3, v4, v5, v6, v7, v8, v9, v10, v11, v12, v13, v14,
             v15, ec_h, ex_h, dst3_h, src3_h, num_h,
             idx_d, idx_s, vr, er, pr, exb, acc,
             semv0, semv1, seme0, seme1, semx0, semx1, sems0, sems1):
        cid = lax.axis_index("c")
        sid = lax.axis_index("s")
        wid = sid * 2 + cid
        vts = (v0, v1, v2, v3, v4, v5, v6, v7, v8, v9, v10, v11, v12, v13,
               v14, v15)
        pltpu.sync_copy(dst3_h.at[:, wid], idx_d)
        pltpu.sync_copy(src3_h.at[:, wid], idx_s)

        for f in range(NCHUNK):
            def zb(j, _):
                for cc in range(VW // 16):
                    pr[0, j, pl.ds(cc * 16, 16)] = jnp.zeros((16,), _f32)
                return 0
            lax.fori_loop(0, EB, zb, 0)
            for r in range(PADN // 16 // EB):
                pltpu.sync_copy(pr.at[0],
                                acc.at[pl.ds(sid * (PADN // 16) + r * EB, EB)])
            plsc.subcore_barrier()

            semv = (semv0, semv1)
            seme = (seme0, seme1)
            semx = (semx0, semx1)

            def issue(g, b):
                e0 = (g * NW + wid) * EB
                pltpu.async_copy(vts[f].at[idx_s.at[g]], vr.at[b], semv[b])
                pltpu.async_copy(ec_h.at[f, pl.ds(e0, EB)], er.at[b], seme[b])
                pltpu.async_copy(ex_h.at[pl.ds(e0, EB)], exb.at[b], semx[b])

            def waitb(b):
                pltpu.make_async_copy(vts[f].at[idx_s.at[0]], vr.at[b],
                                      semv[b]).wait()
                pltpu.make_async_copy(ec_h.at[f, pl.ds(0, EB)], er.at[b],
                                      seme[b]).wait()
                pltpu.make_async_copy(ex_h.at[pl.ds(0, EB)], exb.at[b],
                                      semx[b]).wait()

            semsc = (sems0, sems1)

            def work(g, b, first):
                def edge(j, _):
                    exv = exb[b, j, :]
                    for cc in range(VW // 16):
                        sl = pl.ds(cc * 16, 16)
                        pr[b, j, sl] = (vr[b, j, sl] + er[b, j, sl]) * exv
                    return 0
                if not first:
                    pltpu.make_async_copy(pr.at[b], acc.at[idx_d.at[0]],
                                          semsc[b]).wait()
                lax.fori_loop(0, EB, edge, 0)
                pltpu.async_copy(pr.at[b], acc.at[idx_d.at[g]], semsc[b],
                                 add=True)

            issue(0, 0)
            waitb(0)
            issue(1, 1)
            work(0, 0, True)
            waitb(1)
            issue(2, 0)
            work(1, 1, True)

            def pair(p, _):
                g0 = 2 * p + 2
                waitb(0)
                issue(g0 + 1, 1)
                work(g0, 0, False)
                waitb(1)

                @pl.when(g0 + 2 < NBLK)
                def _():
                    issue(g0 + 2, 0)
                work(g0 + 1, 1, False)
                return 0
            lax.fori_loop(0, (NBLK - 3) // 2, pair, 0)
            waitb(0)
            work(NBLK - 1, 0, False)
            pltpu.make_async_copy(pr.at[0], acc.at[idx_d.at[0]],
                                  semsc[0]).wait()
            pltpu.make_async_copy(pr.at[1], acc.at[idx_d.at[0]],
                                  semsc[1]).wait()

            plsc.subcore_barrier()
            rows = PADN // 16
            pltpu.sync_copy(acc.at[pl.ds(sid * rows, rows)],
                            num_h.at[cid, pl.ds(sid * rows, rows), f])
            plsc.subcore_barrier()

    return kern(*vcs, ec, ex, dst3, src3)


def _finalize(num, den, skip_t, Wc1_p, bc1, Wc2_p, bc2_p, pka_r):
    Bn = 400

    def body(num_ref, den_ref, skip_ref, w1_ref, b1_ref, w2_ref, b2_ref,
             pka_ref, logit_ref, loss_ref):
        i = pl.program_id(0)
        nsum = num_ref[0] + num_ref[1]
        n_t = nsum.reshape(Bn, HID)
        d = den_ref[0] + den_ref[1] + 1e-16
        d_t = jnp.tile(d, (1, DH))
        h_out = n_t / d_t + skip_ref[...]
        hid1 = jnp.maximum(
            jnp.dot(h_out, w1_ref[...], preferred_element_type=_f32) + b1_ref[...],
            0.0)
        logits = jnp.dot(hid1, w2_ref[...], preferred_element_type=_f32) + b2_ref[...]
        logit_ref[...] = logits
        l0 = logits[:, 0:1]
        l1 = logits[:, 1:2]
        m = jnp.maximum(l0, l1)
        lz = m + jnp.log(jnp.exp(l0 - m) + jnp.exp(l1 - m))
        tgt = pka_ref[...] > 0.0
        picked = jnp.where(tgt, l1, l0)
        part = jnp.sum(lz - picked) * (1.0 / N)

        @pl.when(i == 0)
        def _():
            loss_ref[0, 0] = 0.0
        loss_ref[0, 0] += part

    return pl.pallas_call(
        body,
        grid=(N // Bn,),
        in_specs=[
            pl.BlockSpec((2, Bn, NCHUNK, VW), lambda i: (0, i, 0, 0)),
            pl.BlockSpec((2, Bn, HEADS), lambda i: (0, i, 0)),
            pl.BlockSpec((Bn, HID), lambda i: (i, 0)),
            pl.BlockSpec((HID, 256), lambda i: (0, 0)),
            pl.BlockSpec((1, 256), lambda i: (0, 0)),
            pl.BlockSpec((256, 128), lambda i: (0, 0)),
            pl.BlockSpec((1, 128), lambda i: (0, 0)),
            pl.BlockSpec((Bn, 1), lambda i: (i, 0)),
        ],
        out_specs=[
            pl.BlockSpec((Bn, 128), lambda i: (i, 0)),
            pl.BlockSpec(memory_space=pltpu.SMEM),
        ],
        out_shape=[
            jax.ShapeDtypeStruct((N, 128), _f32),
            jax.ShapeDtypeStruct((1, 1), _f32),
        ],
    )(num, den, skip_t, Wc1_p, bc1, Wc2_p, bc2_p, pka_r)


def kernel(x, edge_index, edge_attr, metal_id, batch_vec, pka_labels,
           metal_table, Wp, bp, Wq, bq, Wk, bk, Wv, bv, We, be,
           Wskip, bskip, Wc1, bc1, Wc2, bc2):
    perm = _PERM_T
    inv_scale = 1.0 / np.sqrt(DH)

    Wpx = Wp[:NODE_DIM]
    m_row = metal_table[metal_id]
    mrow_c = (jnp.zeros((8, MEMB), _f32).at[0:1].set(m_row))
    Wq_t = (Wq * inv_scale)[:, perm]
    bq_t = (bq * inv_scale)[perm][None, :]
    Wk_t = Wk[:, perm]
    bk_t = bk[perm][None, :]
    Wv_t = Wv[:, perm]
    bv_t = bv[perm][None, :]
    Wsk_t = Wskip[:, perm]
    bsk_t = bskip[perm][None, :]
    We_r = We[:, perm].reshape(BOND_DIM, NCHUNK, VW).transpose(1, 0, 2)
    be_r = be[perm].reshape(NCHUNK, 1, VW)
    Wc1_p = Wc1[perm, :]
    Wc2_p = jnp.zeros((256, 128), _f32).at[:, :2].set(Wc2)
    bc2_p = jnp.zeros((1, 128), _f32).at[0, :2].set(bc2)

    def mproj_body(m_ref, w_ref, o_ref):
        o_ref[...] = jnp.dot(m_ref[...], w_ref[...], preferred_element_type=_f32)

    mcontrib = pl.pallas_call(
        mproj_body,
        grid=(1,),
        in_specs=[pl.BlockSpec((8, MEMB), lambda i: (0, 0)),
                  pl.BlockSpec((MEMB, HID), lambda i: (0, 0))],
        out_specs=pl.BlockSpec((8, HID), lambda i: (0, 0)),
        out_shape=jax.ShapeDtypeStruct((8, HID), _f32),
    )(mrow_c, Wp[NODE_DIM:])[0:1]

    h = _node_proj(x, Wpx, mcontrib, bp[None, :])
    qt, kt = _two_mm(h, Wq_t, bq_t, Wk_t, bk_t)
    vt, skip_t = _two_mm(h, Wv_t, bv_t, Wsk_t, bsk_t)
    ec = _edge_proj(edge_attr, We_r, be_r)
    vcs = _vchunks(vt)

    src3 = edge_index[0].reshape(NBLK, NW, EB)
    dst3 = edge_index[1].reshape(NBLK, NW, EB)
    ex, den = _sc_pass_a(qt, kt, ec, dst3, src3)
    num = _sc_pass_b(vcs, ec, ex, dst3, src3)

    logits128, loss_acc = _finalize(
        num[:, :N], den[:, :N], skip_t, Wc1_p, bc1[None, :], Wc2_p, bc2_p,
        pka_labels[:, None])

    return (logits128[:, :2], loss_acc[0, 0])
